# baseline (speedup 1.0000x reference)
import jax
import jax.numpy as jnp
from jax import lax
from jax.experimental import pallas as pl
from jax.experimental.pallas import tpu as pltpu
from jax.experimental.pallas import tpu_sc as plsc

BATCH = 1024
SEQ = 80
DIM = 256
LANES = 16
NC = 2
NS = 16
NW = NC * NS
TOK_PER_W = BATCH * SEQ // NW
CHUNK = 40
NCHK = TOK_PER_W // CHUNK
NBUF = 8
GAHEAD = 4


def _body(x_hbm, tok_hbm, pos_hbm, out_hbm, idx_v, pos_v, bufs, gsems, ssems,
          psem):
    wid = lax.axis_index("s") * NC + lax.axis_index("c")
    tokbase = wid * TOK_PER_W
    rowbase = pl.multiple_of(wid * NCHK, 8)

    pltpu.sync_copy(x_hbm.at[pl.ds(rowbase, NCHK)], idx_v)
    pos_copy = pltpu.async_copy(pos_hbm, pos_v, psem)

    def out_at(g):
        return out_hbm.at[pl.ds(pl.multiple_of(tokbase + g * CHUNK, 8), CHUNK)]

    def gather(g, b):
        return pltpu.async_copy(tok_hbm.at[idx_v.at[g]], bufs[b], gsems[b])

    def store(g, b):
        return pltpu.async_copy(bufs[b], out_at(g), ssems[b])

    def gather_wait(g, b):
        pltpu.make_async_copy(tok_hbm.at[idx_v.at[g]], bufs[b], gsems[b]).wait()

    def store_wait(g, b):
        pltpu.make_async_copy(bufs[b], out_at(g), ssems[b]).wait()

    def add_pos(b, off):
        def add_row(r, c):
            for cc in range(DIM // LANES):
                sl = pl.ds(cc * LANES, LANES)
                bufs[b][r, sl] = bufs[b][r, sl] + pos_v[off + r, sl]
            return c

        lax.fori_loop(0, CHUNK, add_row, 0, unroll=2)

    for g in range(GAHEAD):
        gather(g, g)
    pos_copy.wait()

    def group(k, c):
        gb = k * NBUF
        for db in range(NBUF):
            g = gb + db
            bn = (db + GAHEAD) % NBUF
            nxt = g + GAHEAD

            @pl.when(jnp.logical_and(nxt >= NBUF, nxt < NCHK))
            def _():
                store_wait(nxt - NBUF, bn)

            @pl.when(nxt < NCHK)
            def _():
                gather(nxt, bn)

            gather_wait(g, db)
            add_pos(db, (db % 2) * CHUNK)
            store(g, db)
        return c

    lax.fori_loop(0, NCHK // NBUF, group, 0)

    for g in range(NCHK - NBUF, NCHK):
        store_wait(g, g % NBUF)


@jax.jit
def _embed(x2, token_table, pos_table):
    mesh = plsc.VectorSubcoreMesh(core_axis_name="c", subcore_axis_name="s",
                                  num_cores=NC, num_subcores=NS)
    return pl.kernel(
        _body,
        out_type=jax.ShapeDtypeStruct((BATCH * SEQ, DIM), jnp.float32),
        mesh=mesh,
        scratch_types=[
            pltpu.VMEM((NCHK, CHUNK), jnp.int32),
            pltpu.VMEM((SEQ, DIM), jnp.float32),
            [pltpu.VMEM((CHUNK, DIM), jnp.float32) for _ in range(NBUF)],
            [pltpu.SemaphoreType.DMA for _ in range(NBUF)],
            [pltpu.SemaphoreType.DMA for _ in range(NBUF)],
            pltpu.SemaphoreType.DMA,
        ],
    )(x2, token_table, pos_table)


def kernel(x, token_table, pos_table):
    if x.dtype != jnp.int32:
        x = x.astype(jnp.int32)
    out = _embed(x.reshape(BATCH * SEQ // CHUNK, CHUNK), token_table, pos_table)
    return out.reshape(BATCH, SEQ, DIM)


# speedup vs baseline: 1.2676x; 1.2676x over previous
import jax
import jax.numpy as jnp
from jax import lax
from jax.experimental import pallas as pl
from jax.experimental.pallas import tpu as pltpu
from jax.experimental.pallas import tpu_sc as plsc

BATCH = 1024
SEQ = 80
DIM = 256
LANES = 16
NC = 2
NS = 16
NW = NC * NS
ROWS_PER_W = BATCH // NW
NBUF = 4
NSLOT = 1
GAHEAD = 2


def _body(x_hbm, tok_hbm, pos_hbm, out_hbm, idx_v, pos_v, bufs, sp,
          gsems, csems, dsems, psem):
    sid = lax.axis_index("s")
    wid = sid * NC + lax.axis_index("c")
    base = pl.multiple_of(wid * ROWS_PER_W, ROWS_PER_W)
    tokbase = wid * ROWS_PER_W * SEQ

    def out_at(g):
        return out_hbm.at[pl.ds(pl.multiple_of(tokbase + g * SEQ, 8), SEQ)]

    pltpu.sync_copy(x_hbm.at[pl.ds(base, ROWS_PER_W)], idx_v)
    pos_copy = pltpu.async_copy(pos_hbm, pos_v, psem)

    def gather(g, b):
        return pltpu.async_copy(tok_hbm.at[idx_v.at[g]], bufs[b], gsems[b])

    def gather_wait(g, b):
        pltpu.make_async_copy(tok_hbm.at[idx_v.at[g]], bufs[b], gsems[b]).wait()

    def spcopy(b, sl):
        return pltpu.async_copy(bufs[b], sp.at[sid, sl], csems[sl])

    def spcopy_wait(b, sl):
        pltpu.make_async_copy(bufs[b], sp.at[sid, sl], csems[sl]).wait()

    def dma(g, sl):
        return pltpu.async_copy(sp.at[sid, sl], out_at(g), dsems[sl])

    def dma_wait(g, sl):
        pltpu.make_async_copy(sp.at[sid, sl], out_at(g), dsems[sl]).wait()

    def add_pos(b):
        def add_row(r, c):
            for cc in range(DIM // LANES):
                sl = pl.ds(cc * LANES, LANES)
                bufs[b][r, sl] = bufs[b][r, sl] + pos_v[r, sl]
            return c

        lax.fori_loop(0, SEQ, add_row, 0, unroll=2)

    for g in range(GAHEAD):
        gather(g, g)
    pos_copy.wait()

    def group(k, c):
        gb = k * NBUF
        for db in range(NBUF):
            g = gb + db
            bn = (db + GAHEAD) % NBUF
            nxt = g + GAHEAD

            @pl.when(nxt < ROWS_PER_W)
            def _():
                gather(nxt, bn)

            gather_wait(g, db)
            add_pos(db)

            @pl.when(g >= 1)
            def _():
                dma_wait(g - 1, 0)

            spcopy(db, 0)
            spcopy_wait(db, 0)
            dma(g, 0)
        return c

    lax.fori_loop(0, ROWS_PER_W // NBUF, group, 0)

    dma_wait(ROWS_PER_W - 1, 0)


@jax.jit
def _embed(x, token_table, pos_table):
    mesh = plsc.VectorSubcoreMesh(core_axis_name="c", subcore_axis_name="s",
                                  num_cores=NC, num_subcores=NS)
    return pl.kernel(
        _body,
        out_type=jax.ShapeDtypeStruct((BATCH * SEQ, DIM), jnp.float32),
        mesh=mesh,
        scratch_types=[
            pltpu.VMEM((ROWS_PER_W, SEQ), jnp.int32),
            pltpu.VMEM((SEQ, DIM), jnp.float32),
            [pltpu.VMEM((SEQ, DIM), jnp.float32) for _ in range(NBUF)],
            pltpu.VMEM_SHARED((NS, NSLOT, SEQ, DIM), jnp.float32),
            [pltpu.SemaphoreType.DMA for _ in range(NBUF)],
            [pltpu.SemaphoreType.DMA for _ in range(NSLOT)],
            [pltpu.SemaphoreType.DMA for _ in range(NSLOT)],
            pltpu.SemaphoreType.DMA,
        ],
    )(x, token_table, pos_table)


def kernel(x, token_table, pos_table):
    if x.dtype != jnp.int32:
        x = x.astype(jnp.int32)
    out = _embed(x, token_table, pos_table)
    return out.reshape(BATCH, SEQ, DIM)


# speedup vs baseline: 1.5449x; 1.2187x over previous
import functools

import jax
import jax.numpy as jnp
from jax import lax
from jax.experimental import pallas as pl
from jax.experimental.pallas import tpu as pltpu
from jax.experimental.pallas import tpu_sc as plsc

BATCH = 1024
SEQ = 80
DIM = 256
LANES = 16
NC = 2
NS = 16
NW = NC * NS
ROWS_PER_W = BATCH // NW
NBUF = 5
GAHEAD = 2


def _body(x_hbm, tok_hbm, pos_hbm, out_hbm, idx_v, pos_v, bufs, gsems, ssems,
          psem):
    wid = lax.axis_index("s") * NC + lax.axis_index("c")
    base = pl.multiple_of(wid * ROWS_PER_W, ROWS_PER_W)

    pltpu.sync_copy(x_hbm.at[pl.ds(base, ROWS_PER_W)], idx_v)
    pos_copy = pltpu.async_copy(pos_hbm, pos_v, psem)

    def gather(g, b):
        return pltpu.async_copy(tok_hbm.at[idx_v.at[g]], bufs[b], gsems[b])

    def store(g, b):
        return pltpu.async_copy(bufs[b], out_hbm.at[base + g], ssems[b])

    def gather_wait(g, b):
        pltpu.make_async_copy(tok_hbm.at[idx_v.at[g]], bufs[b], gsems[b]).wait()

    def store_wait(g, b):
        pltpu.make_async_copy(bufs[b], out_hbm.at[base + g], ssems[b]).wait()

    def add_pos(b):
        def add_row(r, c):
            for cc in range(DIM // LANES):
                sl = pl.ds(cc * LANES, LANES)
                bufs[b][r, sl] = bufs[b][r, sl] + pos_v[r, sl]
            return c

        lax.fori_loop(0, SEQ, add_row, 0, unroll=2)

    for g in range(GAHEAD):
        gather(g, g)
    pos_copy.wait()

    n_groups = -(-ROWS_PER_W // NBUF)

    def group(k, c):
        gb = k * NBUF
        for db in range(NBUF):
            g = gb + db
            bn = (db + GAHEAD) % NBUF
            nxt = g + GAHEAD

            @pl.when(jnp.logical_and(nxt >= NBUF, nxt < ROWS_PER_W))
            def _():
                store_wait(nxt - NBUF, bn)

            @pl.when(nxt < ROWS_PER_W)
            def _():
                gather(nxt, bn)

            @pl.when(g < ROWS_PER_W)
            def _():
                gather_wait(g, db)
                add_pos(db)
                store(g, db)
        return c

    lax.fori_loop(0, n_groups, group, 0)

    for g in range(ROWS_PER_W - NBUF, ROWS_PER_W):
        store_wait(g, g % NBUF)


@jax.jit
def _embed(x, token_table, pos_table):
    mesh = plsc.VectorSubcoreMesh(core_axis_name="c", subcore_axis_name="s",
                                  num_cores=NC, num_subcores=NS)
    return pl.kernel(
        _body,
        out_type=jax.ShapeDtypeStruct((BATCH, SEQ, DIM), jnp.float32),
        mesh=mesh,
        scratch_types=[
            pltpu.VMEM((ROWS_PER_W, SEQ), jnp.int32),
            pltpu.VMEM((SEQ, DIM), jnp.float32),
            [pltpu.VMEM((SEQ, DIM), jnp.float32) for _ in range(NBUF)],
            [pltpu.SemaphoreType.DMA for _ in range(NBUF)],
            [pltpu.SemaphoreType.DMA for _ in range(NBUF)],
            pltpu.SemaphoreType.DMA,
        ],
    )(x, token_table, pos_table)


def kernel(x, token_table, pos_table):
    if x.dtype != jnp.int32:
        x = x.astype(jnp.int32)
    return _embed(x, token_table, pos_table)


# speedup vs baseline: 1.5536x; 1.0057x over previous
import jax
import jax.numpy as jnp
from jax import lax
from jax.experimental import pallas as pl
from jax.experimental.pallas import tpu as pltpu
from jax.experimental.pallas import tpu_sc as plsc

BATCH = 1024
SEQ = 80
DIM = 256
LANES = 16
NC = 2
NS = 16
NW = NC * NS
ROWS_PER_W = BATCH // NW
NBUF = 4
GAHEAD = 2


def _body(x_hbm, tok_hbm, pos_hbm, out_hbm, idx_v, pos_v, bufs, gsems, ssems,
          psem):
    wid = lax.axis_index("s") * NC + lax.axis_index("c")
    base = pl.multiple_of(wid * ROWS_PER_W, ROWS_PER_W)

    pltpu.sync_copy(x_hbm.at[pl.ds(base, ROWS_PER_W)], idx_v)
    pos_copy = pltpu.async_copy(pos_hbm, pos_v, psem)

    def gather(g, b):
        return pltpu.async_copy(tok_hbm.at[idx_v.at[g]], bufs[b], gsems[b])

    def store(g, b):
        return pltpu.async_copy(bufs[b], out_hbm.at[base + g], ssems[b])

    def gather_wait(g, b):
        pltpu.make_async_copy(tok_hbm.at[idx_v.at[g]], bufs[b], gsems[b]).wait()

    def store_wait(g, b):
        pltpu.make_async_copy(bufs[b], out_hbm.at[base + g], ssems[b]).wait()

    def add_pos(b):
        def add_row(r, c):
            for cc in range(DIM // LANES):
                sl = pl.ds(cc * LANES, LANES)
                bufs[b][r, sl] = bufs[b][r, sl] + pos_v[r, sl]
            return c

        lax.fori_loop(0, SEQ, add_row, 0, unroll=2)

    for g in range(GAHEAD):
        gather(g, g)
    pos_copy.wait()

    def group(k, c):
        gb = k * NBUF
        for db in range(NBUF):
            g = gb + db
            bn = (db + GAHEAD) % NBUF
            nxt = g + GAHEAD

            @pl.when(jnp.logical_and(nxt >= NBUF, nxt < ROWS_PER_W))
            def _():
                store_wait(nxt - NBUF, bn)

            @pl.when(nxt < ROWS_PER_W)
            def _():
                gather(nxt, bn)

            gather_wait(g, db)
            add_pos(db)
            store(g, db)
        return c

    lax.fori_loop(0, ROWS_PER_W // NBUF, group, 0)

    for g in range(ROWS_PER_W - NBUF, ROWS_PER_W):
        store_wait(g, g % NBUF)


@jax.jit
def _embed(x, token_table, pos_table):
    mesh = plsc.VectorSubcoreMesh(core_axis_name="c", subcore_axis_name="s",
                                  num_cores=NC, num_subcores=NS)
    return pl.kernel(
        _body,
        out_type=jax.ShapeDtypeStruct((BATCH, SEQ, DIM), jnp.float32),
        mesh=mesh,
        scratch_types=[
            pltpu.VMEM((ROWS_PER_W, SEQ), jnp.int32),
            pltpu.VMEM((SEQ, DIM), jnp.float32),
            [pltpu.VMEM((SEQ, DIM), jnp.float32) for _ in range(NBUF)],
            [pltpu.SemaphoreType.DMA for _ in range(NBUF)],
            [pltpu.SemaphoreType.DMA for _ in range(NBUF)],
            pltpu.SemaphoreType.DMA,
        ],
    )(x, token_table, pos_table)


def kernel(x, token_table, pos_table):
    if x.dtype != jnp.int32:
        x = x.astype(jnp.int32)
    return _embed(x, token_table, pos_table)


# speedup vs baseline: 1.5706x; 1.0109x over previous
import jax
import jax.numpy as jnp
from jax import lax
from jax.experimental import pallas as pl
from jax.experimental.pallas import tpu as pltpu
from jax.experimental.pallas import tpu_sc as plsc

BATCH = 1024
SEQ = 80
DIM = 256
LANES = 16
NC = 2
NS = 16
NW = NC * NS
ROWS_PER_W = BATCH // NW
NBUF = 4
GAHEAD = 2


def _body(x_hbm, tok_hbm, pos_hbm, out_hbm, idx_v, pos_v, bufs, gsems, ssems,
          psem):
    wid = lax.axis_index("s") * NC + lax.axis_index("c")
    base = pl.multiple_of(wid * ROWS_PER_W, ROWS_PER_W)

    pltpu.sync_copy(x_hbm.at[pl.ds(base, ROWS_PER_W)], idx_v)
    pos_copy = pltpu.async_copy(pos_hbm, pos_v, psem)

    def gather(g, b):
        return pltpu.async_copy(tok_hbm.at[idx_v.at[g]], bufs[b], gsems[b])

    def store(g, b):
        return pltpu.async_copy(bufs[b], out_hbm.at[base + g], ssems[b])

    def gather_wait(g, b):
        pltpu.make_async_copy(tok_hbm.at[idx_v.at[g]], bufs[b], gsems[b]).wait()

    def store_wait(g, b):
        pltpu.make_async_copy(bufs[b], out_hbm.at[base + g], ssems[b]).wait()

    def add_pos(b):
        def add_row(r, c):
            for cc in range(DIM // LANES):
                sl = pl.ds(cc * LANES, LANES)
                bufs[b][r, sl] = bufs[b][r, sl] + pos_v[r, sl]
            return c

        lax.fori_loop(0, SEQ, add_row, 0)

    for g in range(GAHEAD):
        gather(g, g)
    pos_copy.wait()

    def group(k, c):
        gb = k * NBUF
        for db in range(NBUF):
            g = gb + db
            bn = (db + GAHEAD) % NBUF
            nxt = g + GAHEAD

            @pl.when(jnp.logical_and(nxt >= NBUF, nxt < ROWS_PER_W))
            def _():
                store_wait(nxt - NBUF, bn)

            @pl.when(nxt < ROWS_PER_W)
            def _():
                gather(nxt, bn)

            gather_wait(g, db)
            add_pos(db)
            store(g, db)
        return c

    lax.fori_loop(0, ROWS_PER_W // NBUF, group, 0)

    for g in range(ROWS_PER_W - NBUF, ROWS_PER_W):
        store_wait(g, g % NBUF)


@jax.jit
def _embed(x, token_table, pos_table):
    mesh = plsc.VectorSubcoreMesh(core_axis_name="c", subcore_axis_name="s",
                                  num_cores=NC, num_subcores=NS)
    return pl.kernel(
        _body,
        out_type=jax.ShapeDtypeStruct((BATCH, SEQ, DIM), jnp.float32),
        mesh=mesh,
        scratch_types=[
            pltpu.VMEM((ROWS_PER_W, SEQ), jnp.int32),
            pltpu.VMEM((SEQ, DIM), jnp.float32),
            [pltpu.VMEM((SEQ, DIM), jnp.float32) for _ in range(NBUF)],
            [pltpu.SemaphoreType.DMA for _ in range(NBUF)],
            [pltpu.SemaphoreType.DMA for _ in range(NBUF)],
            pltpu.SemaphoreType.DMA,
        ],
    )(x, token_table, pos_table)


def kernel(x, token_table, pos_table):
    if x.dtype != jnp.int32:
        x = x.astype(jnp.int32)
    return _embed(x, token_table, pos_table)
